# baseline (speedup 1.0000x reference)
import jax
import jax.numpy as jnp
from jax import lax
from jax.experimental import pallas as pl
from jax.experimental.pallas import tpu as pltpu
from jax.experimental.pallas import tpu_sc as plsc

N = 10000
E = 320000
D_ATOM = 128
D_BOND = 16
HIDDEN = 128

NC = 2
NS = 16
NW = NC * NS

CHUNK = 128
CPT = 80
PHASE = 40
E_PAD = CHUNK * CPT * NW
E_CHUNKS = E // CHUNK
IDX_ROWS = E_PAD // CHUNK
N_ACC = N
ZROWS = N_ACC // NS
NBUF = 2

_mesh = plsc.VectorSubcoreMesh(
    core_axis_name="c", subcore_axis_name="s", num_cores=NC, num_subcores=NS)


def _zero_acc(zbuf, acc, sid, ncol):
  z = jnp.zeros((16,), jnp.float32)
  @pl.loop(0, CHUNK)
  def _(i):
    for j in range(ncol // 16):
      zbuf[i, pl.ds(j * 16, 16)] = z
  zbase = sid * ZROWS
  for k in range(ZROWS // CHUNK):
    pltpu.sync_copy(zbuf, acc.at[pl.ds(zbase + k * CHUNK, CHUNK)])
  rem = ZROWS % CHUNK
  if rem:
    pltpu.sync_copy(zbuf.at[pl.ds(0, rem)],
                    acc.at[pl.ds(zbase + (ZROWS // CHUNK) * CHUNK, rem)])


def _sc_gather_segsum(h_hbm, src_hbm, dst_hbm, out_hbm,
                      sidx, didx, rows, acc, gsems, ssems):
  cid = lax.axis_index("c")
  sid = lax.axis_index("s")
  wid = cid * NS + sid
  rbase = wid * CPT
  nlive = jnp.minimum(CPT, jnp.maximum(E_CHUNKS - rbase, 0)).astype(jnp.int32)

  def gather(chunk, b):
    pltpu.async_copy(h_hbm.at[sidx.at[chunk]], rows.at[b], gsems.at[b])

  def gather_wait(b):
    pltpu.make_async_copy(h_hbm.at[sidx.at[0]], rows.at[b], gsems.at[b]).wait()

  def scatter(chunk, b):
    pltpu.async_copy(rows.at[b], acc.at[didx.at[chunk]], ssems.at[b], add=True)

  def scatter_wait(b):
    pltpu.make_async_copy(rows.at[b], acc.at[didx.at[0]], ssems.at[b]).wait()

  pltpu.sync_copy(src_hbm.at[pl.ds(rbase, PHASE)], sidx)
  pltpu.sync_copy(dst_hbm.at[pl.ds(rbase, PHASE)], didx)
  gather(1, 1)
  _zero_acc(rows.at[0], acc, sid, HIDDEN)
  gather(0, 0)

  plsc.subcore_barrier()

  def pipeline(nchunks):
    @pl.loop(0, nchunks - NBUF, step=NBUF)
    def _(j):
      for b in range(NBUF):
        gather_wait(b)
        scatter(j + b, b)
      for b in range(NBUF):
        scatter_wait(b)
        gather(j + NBUF + b, b)
    jlast = nchunks - NBUF
    for b in range(NBUF):
      gather_wait(b)
      scatter(jlast + b, b)
    for b in range(NBUF):
      scatter_wait(b)

  np0 = jnp.minimum(PHASE, nlive)
  pipeline(np0)

  np1 = nlive - np0
  @pl.when(np1 > 0)
  def _():
    pltpu.sync_copy(src_hbm.at[pl.ds(rbase + PHASE, PHASE)], sidx)
    pltpu.sync_copy(dst_hbm.at[pl.ds(rbase + PHASE, PHASE)], didx)
    for b in range(NBUF):
      gather(b, b)
    pipeline(np1)

  plsc.subcore_barrier()

  obase = sid * ZROWS
  pltpu.sync_copy(acc.at[pl.ds(obase, ZROWS)],
                  out_hbm.at[cid, pl.ds(obase, ZROWS)])


_gather_segsum = pl.kernel(
    _sc_gather_segsum,
    out_type=jax.ShapeDtypeStruct((NC, N, HIDDEN), jnp.float32),
    mesh=_mesh,
    compiler_params=pltpu.CompilerParams(use_tc_tiling_on_sc=False),
    scratch_types=[
        pltpu.VMEM((PHASE, CHUNK), jnp.int32),
        pltpu.VMEM((PHASE, CHUNK), jnp.int32),
        pltpu.VMEM((NBUF, CHUNK, HIDDEN), jnp.float32),
        pltpu.VMEM_SHARED((N_ACC, HIDDEN), jnp.float32),
        pltpu.SemaphoreType.DMA((NBUF,)),
        pltpu.SemaphoreType.DMA((NBUF,)),
    ],
)


def _sc_edge_segsum(ea_hbm, dst_hbm, out_hbm, didx, rows, acc, gsems, ssems):
  cid = lax.axis_index("c")
  sid = lax.axis_index("s")
  wid = cid * NS + sid
  rbase = wid * CPT
  ebase = wid * CPT * CHUNK
  nlive = jnp.minimum(CPT, jnp.maximum(E_CHUNKS - rbase, 0)).astype(jnp.int32)

  def fetch(chunk, b, p):
    pltpu.async_copy(
        ea_hbm.at[pl.ds(ebase + (p * PHASE + chunk) * CHUNK, CHUNK)],
        rows.at[b], gsems.at[b])

  def fetch_wait(b):
    pltpu.make_async_copy(ea_hbm.at[pl.ds(0, CHUNK)], rows.at[b],
                          gsems.at[b]).wait()

  def scatter(chunk, b):
    pltpu.async_copy(rows.at[b], acc.at[didx.at[chunk]], ssems.at[b], add=True)

  def scatter_wait(b):
    pltpu.make_async_copy(rows.at[b], acc.at[didx.at[0]], ssems.at[b]).wait()

  pltpu.sync_copy(dst_hbm.at[pl.ds(rbase, PHASE)], didx)
  fetch(1, 1, 0)
  _zero_acc(rows.at[0], acc, sid, D_BOND)
  fetch(0, 0, 0)

  plsc.subcore_barrier()

  def pipeline(nchunks, p):
    @pl.loop(0, nchunks - NBUF, step=NBUF)
    def _(j):
      for b in range(NBUF):
        fetch_wait(b)
        scatter(j + b, b)
      for b in range(NBUF):
        scatter_wait(b)
        fetch(j + NBUF + b, b, p)
    jlast = nchunks - NBUF
    for b in range(NBUF):
      fetch_wait(b)
      scatter(jlast + b, b)
    for b in range(NBUF):
      scatter_wait(b)

  np0 = jnp.minimum(PHASE, nlive)
  pipeline(np0, 0)

  np1 = nlive - np0
  @pl.when(np1 > 0)
  def _():
    pltpu.sync_copy(dst_hbm.at[pl.ds(rbase + PHASE, PHASE)], didx)
    for b in range(NBUF):
      fetch(b, b, 1)
    pipeline(np1, 1)

  plsc.subcore_barrier()

  obase = sid * ZROWS
  pltpu.sync_copy(acc.at[pl.ds(obase, ZROWS)],
                  out_hbm.at[cid, pl.ds(obase, ZROWS)])


_edge_segsum = pl.kernel(
    _sc_edge_segsum,
    out_type=jax.ShapeDtypeStruct((NC, N, D_BOND), jnp.float32),
    mesh=_mesh,
    compiler_params=pltpu.CompilerParams(use_tc_tiling_on_sc=False),
    scratch_types=[
        pltpu.VMEM((PHASE, CHUNK), jnp.int32),
        pltpu.VMEM((NBUF, CHUNK, D_BOND), jnp.float32),
        pltpu.VMEM_SHARED((N_ACC, D_BOND), jnp.float32),
        pltpu.SemaphoreType.DMA((NBUF,)),
        pltpu.SemaphoreType.DMA((NBUF,)),
    ],
)


_ROWS_BLK = 1000
_GRID = N // _ROWS_BLK

_W_SPEC = lambda r, c: pl.BlockSpec((r, c), lambda i: (0, 0))
_ROW_SPEC = lambda c: pl.BlockSpec((_ROWS_BLK, c), lambda i: (i, 0))
_PAIR_SPEC = lambda c: pl.BlockSpec((NC, _ROWS_BLK, c), lambda i: (0, i, 0))


def _dot(a, b):
  return jnp.dot(a, b, preferred_element_type=jnp.float32)


def _tc_h0_body(x_ref, win_ref, bin_ref, h0_ref):
  h0_ref[...] = jnp.maximum(_dot(x_ref[...], win_ref[...]) + bin_ref[...], 0.0)


def _tc_h0(x, W_in, b_in):
  return pl.pallas_call(
      _tc_h0_body,
      grid=(_GRID,),
      in_specs=[_ROW_SPEC(D_ATOM), _W_SPEC(D_ATOM, HIDDEN), _W_SPEC(1, HIDDEN)],
      out_specs=_ROW_SPEC(HIDDEN),
      out_shape=jax.ShapeDtypeStruct((N, HIDDEN), jnp.float32),
  )(x, W_in, b_in)


def _agg_h(acc_ref, ea_ref, h0_ref, wedge_ref, wh_ref, bh_ref):
  eagg = _dot(ea_ref[0] + ea_ref[1], wedge_ref[...])
  agg = acc_ref[0] + acc_ref[1] + eagg
  return jnp.maximum(_dot(agg, wh_ref[...]) + bh_ref[...] + h0_ref[...], 0.0)


def _tc_step_body(acc_ref, ea_ref, h0_ref, wedge_ref, wh_ref, bh_ref, h_ref):
  h_ref[...] = _agg_h(acc_ref, ea_ref, h0_ref, wedge_ref, wh_ref, bh_ref)


def _tc_step(acc, ea2, h0, W_edge, W_h, b_h):
  return pl.pallas_call(
      _tc_step_body,
      grid=(_GRID,),
      in_specs=[
          _PAIR_SPEC(HIDDEN), _PAIR_SPEC(D_BOND), _ROW_SPEC(HIDDEN),
          _W_SPEC(D_BOND, HIDDEN), _W_SPEC(HIDDEN, HIDDEN), _W_SPEC(1, HIDDEN),
      ],
      out_specs=_ROW_SPEC(HIDDEN),
      out_shape=jax.ShapeDtypeStruct((N, HIDDEN), jnp.float32),
  )(acc, ea2, h0, W_edge, W_h, b_h)


def _tc_step_out_body(acc_ref, ea_ref, h0_ref, x_ref, wedge_ref, wh_ref,
                      bh_ref, wo1_ref, wo2_ref, bo_ref, out_ref):
  h = _agg_h(acc_ref, ea_ref, h0_ref, wedge_ref, wh_ref, bh_ref)
  out_ref[...] = jnp.maximum(
      _dot(x_ref[...], wo1_ref[...]) + _dot(h, wo2_ref[...]) + bo_ref[...],
      0.0)


def _tc_step_out(acc, ea2, h0, x, W_edge, W_h, b_h, W_o1, W_o2, b_o):
  return pl.pallas_call(
      _tc_step_out_body,
      grid=(_GRID,),
      in_specs=[
          _PAIR_SPEC(HIDDEN), _PAIR_SPEC(D_BOND), _ROW_SPEC(HIDDEN),
          _ROW_SPEC(D_ATOM),
          _W_SPEC(D_BOND, HIDDEN), _W_SPEC(HIDDEN, HIDDEN), _W_SPEC(1, HIDDEN),
          _W_SPEC(D_ATOM, HIDDEN), _W_SPEC(HIDDEN, HIDDEN), _W_SPEC(1, HIDDEN),
      ],
      out_specs=_ROW_SPEC(HIDDEN),
      out_shape=jax.ShapeDtypeStruct((N, HIDDEN), jnp.float32),
  )(acc, ea2, h0, x, W_edge, W_h, b_h, W_o1, W_o2, b_o)


@jax.jit
def kernel(x, edge_index, edge_attr, W_in, b_in, W_edge, W_h, b_h, W_o, b_o):
  src = edge_index[0].astype(jnp.int32)
  dst = edge_index[1].astype(jnp.int32)
  pad = E_PAD - E
  zpad = jnp.zeros((pad,), jnp.int32)
  src2d = jnp.concatenate([src, zpad]).reshape(IDX_ROWS, CHUNK)
  dst2d = jnp.concatenate([dst, zpad]).reshape(IDX_ROWS, CHUNK)

  b_in2 = b_in.reshape(1, HIDDEN)
  b_h2 = b_h.reshape(1, HIDDEN)
  b_o2 = b_o.reshape(1, HIDDEN)

  h0 = _tc_h0(x, W_in, b_in2)
  acc = _gather_segsum(h0, src2d, dst2d)
  edge_attr_d, _ = lax.optimization_barrier((edge_attr, h0))
  ea2 = _edge_segsum(edge_attr_d, dst2d)
  h = _tc_step(acc, ea2, h0, W_edge, W_h, b_h2)

  acc = _gather_segsum(h, src2d, dst2d)
  h = _tc_step(acc, ea2, h0, W_edge, W_h, b_h2)

  acc = _gather_segsum(h, src2d, dst2d)
  return _tc_step_out(acc, ea2, h0, x, W_edge, W_h, b_h2,
                      W_o[:D_ATOM], W_o[D_ATOM:], b_o2)


# speedup vs baseline: 1.2559x; 1.2559x over previous
import jax
import jax.numpy as jnp
from jax import lax
from jax.experimental import pallas as pl
from jax.experimental.pallas import tpu as pltpu
from jax.experimental.pallas import tpu_sc as plsc

N = 10000
E = 320000
D_ATOM = 128
D_BOND = 16
HIDDEN = 128

NC = 2
NS = 16
NW = NC * NS

CHUNK = 128
CPT = 80
PHASE = 40
E_PAD = CHUNK * CPT * NW
E_CHUNKS = E // CHUNK
IDX_ROWS = E_PAD // CHUNK
N_ACC = N
ZROWS = N_ACC // NS
NBUF = 2

GCHUNK = 64
GCPT = 160
GPHASE = 40
GROWS = GPHASE // 2
G_CHUNKS = E // GCHUNK
GNBUF = 4

_mesh = plsc.VectorSubcoreMesh(
    core_axis_name="c", subcore_axis_name="s", num_cores=NC, num_subcores=NS)


def _zero_acc(zbuf, acc, sid, nrows, ncol):
  z = jnp.zeros((16,), jnp.float32)
  @pl.loop(0, nrows)
  def _(i):
    for j in range(ncol // 16):
      zbuf[i, pl.ds(j * 16, 16)] = z
  zbase = sid * ZROWS
  for k in range(ZROWS // nrows):
    pltpu.sync_copy(zbuf, acc.at[pl.ds(zbase + k * nrows, nrows)])
  rem = ZROWS % nrows
  if rem:
    pltpu.sync_copy(zbuf.at[pl.ds(0, rem)],
                    acc.at[pl.ds(zbase + (ZROWS // nrows) * nrows, rem)])


def _sc_gather_segsum(h_hbm, src_hbm, dst_hbm, out_hbm,
                      stage, sidx, didx, rows, acc, gsems, ssems):
  cid = lax.axis_index("c")
  sid = lax.axis_index("s")
  wid = cid * NS + sid
  rbase = wid * CPT
  nlive = jnp.minimum(GCPT, jnp.maximum(G_CHUNKS - wid * GCPT, 0))

  def gather(chunk, b):
    pltpu.async_copy(h_hbm.at[sidx.at[chunk]], rows.at[b], gsems.at[b])

  def gather_wait(b):
    pltpu.make_async_copy(h_hbm.at[sidx.at[0]], rows.at[b], gsems.at[b]).wait()

  def scatter(chunk, b):
    pltpu.async_copy(rows.at[b], acc.at[didx.at[chunk]], ssems.at[b], add=True)

  def scatter_wait(b):
    pltpu.make_async_copy(rows.at[b], acc.at[didx.at[0]], ssems.at[b]).wait()

  def stage_phase(p, idx128_hbm, idx64):
    pltpu.sync_copy(idx128_hbm.at[pl.ds(rbase + p * GROWS, GROWS)], stage)
    for r in range(GROWS):
      for half in range(2):
        for k in range(4):
          idx64[2 * r + half, pl.ds(16 * k, 16)] = (
              stage[r, pl.ds(64 * half + 16 * k, 16)])

  def pipeline(nchunks):
    @pl.loop(0, nchunks - GNBUF, step=GNBUF)
    def _(j):
      for b in range(GNBUF):
        gather_wait(b)
        scatter(j + b, b)
      for b in range(GNBUF):
        scatter_wait(b)
        gather(j + GNBUF + b, b)
    jlast = nchunks - GNBUF
    for b in range(GNBUF):
      gather_wait(b)
      scatter(jlast + b, b)
    for b in range(GNBUF):
      scatter_wait(b)

  stage_phase(0, src_hbm, sidx)
  stage_phase(0, dst_hbm, didx)
  for b in range(1, GNBUF):
    gather(b, b)
  _zero_acc(rows.at[0], acc, sid, GCHUNK, HIDDEN)
  gather(0, 0)

  plsc.subcore_barrier()

  np_prev = jnp.minimum(GPHASE, nlive)
  pipeline(np_prev)
  for p in range(1, 4):
    np_p = jnp.minimum(GPHASE, jnp.maximum(nlive - p * GPHASE, 0))
    @pl.when(np_p > 0)
    def _(p=p, np_p=np_p):
      stage_phase(p, src_hbm, sidx)
      stage_phase(p, dst_hbm, didx)
      for b in range(GNBUF):
        gather(b, b)
      pipeline(np_p)

  plsc.subcore_barrier()

  obase = sid * ZROWS
  pltpu.sync_copy(acc.at[pl.ds(obase, ZROWS)],
                  out_hbm.at[cid, pl.ds(obase, ZROWS)])


_gather_segsum = pl.kernel(
    _sc_gather_segsum,
    out_type=jax.ShapeDtypeStruct((NC, N, HIDDEN), jnp.float32),
    mesh=_mesh,
    compiler_params=pltpu.CompilerParams(use_tc_tiling_on_sc=False),
    scratch_types=[
        pltpu.VMEM((GROWS, CHUNK), jnp.int32),
        pltpu.VMEM((GPHASE, GCHUNK), jnp.int32),
        pltpu.VMEM((GPHASE, GCHUNK), jnp.int32),
        pltpu.VMEM((GNBUF, GCHUNK, HIDDEN), jnp.float32),
        pltpu.VMEM_SHARED((N_ACC, HIDDEN), jnp.float32),
        pltpu.SemaphoreType.DMA((GNBUF,)),
        pltpu.SemaphoreType.DMA((GNBUF,)),
    ],
)


def _sc_edge_segsum(ea_hbm, dst_hbm, out_hbm, didx, rows, acc, gsems, ssems):
  cid = lax.axis_index("c")
  sid = lax.axis_index("s")
  wid = cid * NS + sid
  rbase = wid * CPT
  ebase = wid * CPT * CHUNK
  nlive = jnp.minimum(CPT, jnp.maximum(E_CHUNKS - rbase, 0)).astype(jnp.int32)

  def fetch(chunk, b, p):
    pltpu.async_copy(
        ea_hbm.at[pl.ds(ebase + (p * PHASE + chunk) * CHUNK, CHUNK)],
        rows.at[b], gsems.at[b])

  def fetch_wait(b):
    pltpu.make_async_copy(ea_hbm.at[pl.ds(0, CHUNK)], rows.at[b],
                          gsems.at[b]).wait()

  def scatter(chunk, b):
    pltpu.async_copy(rows.at[b], acc.at[didx.at[chunk]], ssems.at[b], add=True)

  def scatter_wait(b):
    pltpu.make_async_copy(rows.at[b], acc.at[didx.at[0]], ssems.at[b]).wait()

  pltpu.sync_copy(dst_hbm.at[pl.ds(rbase, PHASE)], didx)
  fetch(1, 1, 0)
  _zero_acc(rows.at[0], acc, sid, CHUNK, D_BOND)
  fetch(0, 0, 0)

  plsc.subcore_barrier()

  def pipeline(nchunks, p):
    @pl.loop(0, nchunks - NBUF, step=NBUF)
    def _(j):
      for b in range(NBUF):
        fetch_wait(b)
        scatter(j + b, b)
      for b in range(NBUF):
        scatter_wait(b)
        fetch(j + NBUF + b, b, p)
    jlast = nchunks - NBUF
    for b in range(NBUF):
      fetch_wait(b)
      scatter(jlast + b, b)
    for b in range(NBUF):
      scatter_wait(b)

  np0 = jnp.minimum(PHASE, nlive)
  pipeline(np0, 0)

  np1 = nlive - np0
  @pl.when(np1 > 0)
  def _():
    pltpu.sync_copy(dst_hbm.at[pl.ds(rbase + PHASE, PHASE)], didx)
    for b in range(NBUF):
      fetch(b, b, 1)
    pipeline(np1, 1)

  plsc.subcore_barrier()

  obase = sid * ZROWS
  pltpu.sync_copy(acc.at[pl.ds(obase, ZROWS)],
                  out_hbm.at[cid, pl.ds(obase, ZROWS)])


_edge_segsum = pl.kernel(
    _sc_edge_segsum,
    out_type=jax.ShapeDtypeStruct((NC, N, D_BOND), jnp.float32),
    mesh=_mesh,
    compiler_params=pltpu.CompilerParams(use_tc_tiling_on_sc=False),
    scratch_types=[
        pltpu.VMEM((PHASE, CHUNK), jnp.int32),
        pltpu.VMEM((NBUF, CHUNK, D_BOND), jnp.float32),
        pltpu.VMEM_SHARED((N_ACC, D_BOND), jnp.float32),
        pltpu.SemaphoreType.DMA((NBUF,)),
        pltpu.SemaphoreType.DMA((NBUF,)),
    ],
)


_ROWS_BLK = 1000
_GRID = N // _ROWS_BLK

_W_SPEC = lambda r, c: pl.BlockSpec((r, c), lambda i: (0, 0))
_ROW_SPEC = lambda c: pl.BlockSpec((_ROWS_BLK, c), lambda i: (i, 0))
_PAIR_SPEC = lambda c: pl.BlockSpec((NC, _ROWS_BLK, c), lambda i: (0, i, 0))


def _dot(a, b):
  return jnp.dot(a, b, preferred_element_type=jnp.float32)


def _tc_h0_body(x_ref, win_ref, bin_ref, h0_ref):
  h0_ref[...] = jnp.maximum(_dot(x_ref[...], win_ref[...]) + bin_ref[...], 0.0)


def _tc_h0(x, W_in, b_in):
  return pl.pallas_call(
      _tc_h0_body,
      grid=(_GRID,),
      in_specs=[_ROW_SPEC(D_ATOM), _W_SPEC(D_ATOM, HIDDEN), _W_SPEC(1, HIDDEN)],
      out_specs=_ROW_SPEC(HIDDEN),
      out_shape=jax.ShapeDtypeStruct((N, HIDDEN), jnp.float32),
  )(x, W_in, b_in)


def _agg_h(acc_ref, ea_ref, h0_ref, wedge_ref, wh_ref, bh_ref):
  eagg = _dot(ea_ref[0] + ea_ref[1], wedge_ref[...])
  agg = acc_ref[0] + acc_ref[1] + eagg
  return jnp.maximum(_dot(agg, wh_ref[...]) + bh_ref[...] + h0_ref[...], 0.0)


def _tc_step_body(acc_ref, ea_ref, h0_ref, wedge_ref, wh_ref, bh_ref, h_ref):
  h_ref[...] = _agg_h(acc_ref, ea_ref, h0_ref, wedge_ref, wh_ref, bh_ref)


def _tc_step(acc, ea2, h0, W_edge, W_h, b_h):
  return pl.pallas_call(
      _tc_step_body,
      grid=(_GRID,),
      in_specs=[
          _PAIR_SPEC(HIDDEN), _PAIR_SPEC(D_BOND), _ROW_SPEC(HIDDEN),
          _W_SPEC(D_BOND, HIDDEN), _W_SPEC(HIDDEN, HIDDEN), _W_SPEC(1, HIDDEN),
      ],
      out_specs=_ROW_SPEC(HIDDEN),
      out_shape=jax.ShapeDtypeStruct((N, HIDDEN), jnp.float32),
  )(acc, ea2, h0, W_edge, W_h, b_h)


def _tc_step_out_body(acc_ref, ea_ref, h0_ref, x_ref, wedge_ref, wh_ref,
                      bh_ref, wo1_ref, wo2_ref, bo_ref, out_ref):
  h = _agg_h(acc_ref, ea_ref, h0_ref, wedge_ref, wh_ref, bh_ref)
  out_ref[...] = jnp.maximum(
      _dot(x_ref[...], wo1_ref[...]) + _dot(h, wo2_ref[...]) + bo_ref[...],
      0.0)


def _tc_step_out(acc, ea2, h0, x, W_edge, W_h, b_h, W_o1, W_o2, b_o):
  return pl.pallas_call(
      _tc_step_out_body,
      grid=(_GRID,),
      in_specs=[
          _PAIR_SPEC(HIDDEN), _PAIR_SPEC(D_BOND), _ROW_SPEC(HIDDEN),
          _ROW_SPEC(D_ATOM),
          _W_SPEC(D_BOND, HIDDEN), _W_SPEC(HIDDEN, HIDDEN), _W_SPEC(1, HIDDEN),
          _W_SPEC(D_ATOM, HIDDEN), _W_SPEC(HIDDEN, HIDDEN), _W_SPEC(1, HIDDEN),
      ],
      out_specs=_ROW_SPEC(HIDDEN),
      out_shape=jax.ShapeDtypeStruct((N, HIDDEN), jnp.float32),
  )(acc, ea2, h0, x, W_edge, W_h, b_h, W_o1, W_o2, b_o)


@jax.jit
def kernel(x, edge_index, edge_attr, W_in, b_in, W_edge, W_h, b_h, W_o, b_o):
  src = edge_index[0].astype(jnp.int32)
  dst = edge_index[1].astype(jnp.int32)
  pad = E_PAD - E
  zpad = jnp.zeros((pad,), jnp.int32)
  src2d = jnp.concatenate([src, zpad]).reshape(IDX_ROWS, CHUNK)
  dst2d = jnp.concatenate([dst, zpad]).reshape(IDX_ROWS, CHUNK)

  b_in2 = b_in.reshape(1, HIDDEN)
  b_h2 = b_h.reshape(1, HIDDEN)
  b_o2 = b_o.reshape(1, HIDDEN)

  h0 = _tc_h0(x, W_in, b_in2)
  acc = _gather_segsum(h0, src2d, dst2d)
  ea2 = _edge_segsum(edge_attr, dst2d)
  h = _tc_step(acc, ea2, h0, W_edge, W_h, b_h2)

  acc = _gather_segsum(h, src2d, dst2d)
  h = _tc_step(acc, ea2, h0, W_edge, W_h, b_h2)

  acc = _gather_segsum(h, src2d, dst2d)
  return _tc_step_out(acc, ea2, h0, x, W_edge, W_h, b_h2,
                      W_o[:D_ATOM], W_o[D_ATOM:], b_o2)


# speedup vs baseline: 1.2870x; 1.0247x over previous
import jax
import jax.numpy as jnp
from jax import lax
from jax.experimental import pallas as pl
from jax.experimental.pallas import tpu as pltpu
from jax.experimental.pallas import tpu_sc as plsc

N = 10000
E = 320000
D_ATOM = 128
D_BOND = 16
HIDDEN = 128

NC = 2
NS = 16
NW = NC * NS

CHUNK = 128
CPT = 80
PHASE = 40
E_PAD = CHUNK * CPT * NW
E_CHUNKS = E // CHUNK
IDX_ROWS = E_PAD // CHUNK
N_ACC = N
ZROWS = N_ACC // NS
NBUF = 4

GCHUNK = 64
GCPT = 160
GPHASE = 40
GROWS = GPHASE // 2
G_CHUNKS = E // GCHUNK
GNBUF = 4

_mesh = plsc.VectorSubcoreMesh(
    core_axis_name="c", subcore_axis_name="s", num_cores=NC, num_subcores=NS)


def _zero_acc(zbuf, acc, sid, nrows, ncol):
  z = jnp.zeros((16,), jnp.float32)
  @pl.loop(0, nrows)
  def _(i):
    for j in range(ncol // 16):
      zbuf[i, pl.ds(j * 16, 16)] = z
  zbase = sid * ZROWS
  for k in range(ZROWS // nrows):
    pltpu.sync_copy(zbuf, acc.at[pl.ds(zbase + k * nrows, nrows)])
  rem = ZROWS % nrows
  if rem:
    pltpu.sync_copy(zbuf.at[pl.ds(0, rem)],
                    acc.at[pl.ds(zbase + (ZROWS // nrows) * nrows, rem)])


def _sc_gather_segsum(h_hbm, src_hbm, dst_hbm, out_hbm,
                      stage, sidx, didx, rows, acc, gsems, ssems):
  cid = lax.axis_index("c")
  sid = lax.axis_index("s")
  wid = cid * NS + sid
  rbase = wid * CPT
  nlive = jnp.minimum(GCPT, jnp.maximum(G_CHUNKS - wid * GCPT, 0))

  def gather(chunk, b):
    pltpu.async_copy(h_hbm.at[sidx.at[chunk]], rows.at[b], gsems.at[b])

  def gather_wait(b):
    pltpu.make_async_copy(h_hbm.at[sidx.at[0]], rows.at[b], gsems.at[b]).wait()

  def scatter(chunk, b):
    pltpu.async_copy(rows.at[b], acc.at[didx.at[chunk]], ssems.at[b], add=True)

  def scatter_wait(b):
    pltpu.make_async_copy(rows.at[b], acc.at[didx.at[0]], ssems.at[b]).wait()

  def stage_phase(p, idx128_hbm, idx64):
    pltpu.sync_copy(idx128_hbm.at[pl.ds(rbase + p * GROWS, GROWS)], stage)
    for r in range(GROWS):
      for half in range(2):
        for k in range(4):
          idx64[2 * r + half, pl.ds(16 * k, 16)] = (
              stage[r, pl.ds(64 * half + 16 * k, 16)])

  def pipeline(nchunks):
    @pl.loop(0, nchunks - GNBUF, step=GNBUF)
    def _(j):
      for b in range(GNBUF):
        gather_wait(b)
        scatter(j + b, b)
      for b in range(GNBUF):
        scatter_wait(b)
        gather(j + GNBUF + b, b)
    jlast = nchunks - GNBUF
    for b in range(GNBUF):
      gather_wait(b)
      scatter(jlast + b, b)
    for b in range(GNBUF):
      scatter_wait(b)

  stage_phase(0, src_hbm, sidx)
  stage_phase(0, dst_hbm, didx)
  for b in range(1, GNBUF):
    gather(b, b)
  _zero_acc(rows.at[0], acc, sid, GCHUNK, HIDDEN)
  gather(0, 0)

  plsc.subcore_barrier()

  np_prev = jnp.minimum(GPHASE, nlive)
  pipeline(np_prev)
  for p in range(1, 4):
    np_p = jnp.minimum(GPHASE, jnp.maximum(nlive - p * GPHASE, 0))
    @pl.when(np_p > 0)
    def _(p=p, np_p=np_p):
      stage_phase(p, src_hbm, sidx)
      stage_phase(p, dst_hbm, didx)
      for b in range(GNBUF):
        gather(b, b)
      pipeline(np_p)

  plsc.subcore_barrier()

  obase = sid * ZROWS
  pltpu.sync_copy(acc.at[pl.ds(obase, ZROWS)],
                  out_hbm.at[cid, pl.ds(obase, ZROWS)])


_gather_segsum = pl.kernel(
    _sc_gather_segsum,
    out_type=jax.ShapeDtypeStruct((NC, N, HIDDEN), jnp.float32),
    mesh=_mesh,
    compiler_params=pltpu.CompilerParams(use_tc_tiling_on_sc=False),
    scratch_types=[
        pltpu.VMEM((GROWS, CHUNK), jnp.int32),
        pltpu.VMEM((GPHASE, GCHUNK), jnp.int32),
        pltpu.VMEM((GPHASE, GCHUNK), jnp.int32),
        pltpu.VMEM((GNBUF, GCHUNK, HIDDEN), jnp.float32),
        pltpu.VMEM_SHARED((N_ACC, HIDDEN), jnp.float32),
        pltpu.SemaphoreType.DMA((GNBUF,)),
        pltpu.SemaphoreType.DMA((GNBUF,)),
    ],
)


def _sc_edge_segsum(ea_hbm, dst_hbm, out_hbm, didx, rows, acc, gsems, ssems):
  cid = lax.axis_index("c")
  sid = lax.axis_index("s")
  wid = cid * NS + sid
  rbase = wid * CPT
  ebase = wid * CPT * CHUNK
  nlive = jnp.minimum(CPT, jnp.maximum(E_CHUNKS - rbase, 0)).astype(jnp.int32)

  def fetch(chunk, b, p):
    pltpu.async_copy(
        ea_hbm.at[pl.ds(ebase + (p * PHASE + chunk) * CHUNK, CHUNK)],
        rows.at[b], gsems.at[b])

  def fetch_wait(b):
    pltpu.make_async_copy(ea_hbm.at[pl.ds(0, CHUNK)], rows.at[b],
                          gsems.at[b]).wait()

  def scatter(chunk, b):
    pltpu.async_copy(rows.at[b], acc.at[didx.at[chunk]], ssems.at[b], add=True)

  def scatter_wait(b):
    pltpu.make_async_copy(rows.at[b], acc.at[didx.at[0]], ssems.at[b]).wait()

  pltpu.sync_copy(dst_hbm.at[pl.ds(rbase, PHASE)], didx)
  for b in range(1, NBUF):
    fetch(b, b, 0)
  _zero_acc(rows.at[0], acc, sid, CHUNK, D_BOND)
  fetch(0, 0, 0)

  plsc.subcore_barrier()

  def pipeline(nchunks, p):
    @pl.loop(0, nchunks - NBUF, step=NBUF)
    def _(j):
      for b in range(NBUF):
        fetch_wait(b)
        scatter(j + b, b)
      for b in range(NBUF):
        scatter_wait(b)
        fetch(j + NBUF + b, b, p)
    jlast = nchunks - NBUF
    for b in range(NBUF):
      fetch_wait(b)
      scatter(jlast + b, b)
    for b in range(NBUF):
      scatter_wait(b)

  np0 = jnp.minimum(PHASE, nlive)
  pipeline(np0, 0)

  np1 = nlive - np0
  @pl.when(np1 > 0)
  def _():
    pltpu.sync_copy(dst_hbm.at[pl.ds(rbase + PHASE, PHASE)], didx)
    for b in range(NBUF):
      fetch(b, b, 1)
    pipeline(np1, 1)

  plsc.subcore_barrier()

  obase = sid * ZROWS
  pltpu.sync_copy(acc.at[pl.ds(obase, ZROWS)],
                  out_hbm.at[cid, pl.ds(obase, ZROWS)])


_edge_segsum = pl.kernel(
    _sc_edge_segsum,
    out_type=jax.ShapeDtypeStruct((NC, N, D_BOND), jnp.float32),
    mesh=_mesh,
    compiler_params=pltpu.CompilerParams(use_tc_tiling_on_sc=False),
    scratch_types=[
        pltpu.VMEM((PHASE, CHUNK), jnp.int32),
        pltpu.VMEM((NBUF, CHUNK, D_BOND), jnp.float32),
        pltpu.VMEM_SHARED((N_ACC, D_BOND), jnp.float32),
        pltpu.SemaphoreType.DMA((NBUF,)),
        pltpu.SemaphoreType.DMA((NBUF,)),
    ],
)


_ROWS_BLK = 1000
_GRID = N // _ROWS_BLK

_W_SPEC = lambda r, c: pl.BlockSpec((r, c), lambda i: (0, 0))
_ROW_SPEC = lambda c: pl.BlockSpec((_ROWS_BLK, c), lambda i: (i, 0))
_PAIR_SPEC = lambda c: pl.BlockSpec((NC, _ROWS_BLK, c), lambda i: (0, i, 0))


def _dot(a, b):
  return jnp.dot(a, b, preferred_element_type=jnp.float32)


def _tc_h0_body(x_ref, win_ref, bin_ref, h0_ref):
  h0_ref[...] = jnp.maximum(_dot(x_ref[...], win_ref[...]) + bin_ref[...], 0.0)


def _tc_h0(x, W_in, b_in):
  return pl.pallas_call(
      _tc_h0_body,
      grid=(_GRID,),
      in_specs=[_ROW_SPEC(D_ATOM), _W_SPEC(D_ATOM, HIDDEN), _W_SPEC(1, HIDDEN)],
      out_specs=_ROW_SPEC(HIDDEN),
      out_shape=jax.ShapeDtypeStruct((N, HIDDEN), jnp.float32),
  )(x, W_in, b_in)


def _agg_h(acc_ref, ea_ref, h0_ref, wedge_ref, wh_ref, bh_ref):
  eagg = _dot(ea_ref[0] + ea_ref[1], wedge_ref[...])
  agg = acc_ref[0] + acc_ref[1] + eagg
  return jnp.maximum(_dot(agg, wh_ref[...]) + bh_ref[...] + h0_ref[...], 0.0)


def _tc_step_body(acc_ref, ea_ref, h0_ref, wedge_ref, wh_ref, bh_ref, h_ref):
  h_ref[...] = _agg_h(acc_ref, ea_ref, h0_ref, wedge_ref, wh_ref, bh_ref)


def _tc_step(acc, ea2, h0, W_edge, W_h, b_h):
  return pl.pallas_call(
      _tc_step_body,
      grid=(_GRID,),
      in_specs=[
          _PAIR_SPEC(HIDDEN), _PAIR_SPEC(D_BOND), _ROW_SPEC(HIDDEN),
          _W_SPEC(D_BOND, HIDDEN), _W_SPEC(HIDDEN, HIDDEN), _W_SPEC(1, HIDDEN),
      ],
      out_specs=_ROW_SPEC(HIDDEN),
      out_shape=jax.ShapeDtypeStruct((N, HIDDEN), jnp.float32),
  )(acc, ea2, h0, W_edge, W_h, b_h)


def _tc_step_out_body(acc_ref, ea_ref, h0_ref, x_ref, wedge_ref, wh_ref,
                      bh_ref, wo1_ref, wo2_ref, bo_ref, out_ref):
  h = _agg_h(acc_ref, ea_ref, h0_ref, wedge_ref, wh_ref, bh_ref)
  out_ref[...] = jnp.maximum(
      _dot(x_ref[...], wo1_ref[...]) + _dot(h, wo2_ref[...]) + bo_ref[...],
      0.0)


def _tc_step_out(acc, ea2, h0, x, W_edge, W_h, b_h, W_o1, W_o2, b_o):
  return pl.pallas_call(
      _tc_step_out_body,
      grid=(_GRID,),
      in_specs=[
          _PAIR_SPEC(HIDDEN), _PAIR_SPEC(D_BOND), _ROW_SPEC(HIDDEN),
          _ROW_SPEC(D_ATOM),
          _W_SPEC(D_BOND, HIDDEN), _W_SPEC(HIDDEN, HIDDEN), _W_SPEC(1, HIDDEN),
          _W_SPEC(D_ATOM, HIDDEN), _W_SPEC(HIDDEN, HIDDEN), _W_SPEC(1, HIDDEN),
      ],
      out_specs=_ROW_SPEC(HIDDEN),
      out_shape=jax.ShapeDtypeStruct((N, HIDDEN), jnp.float32),
  )(acc, ea2, h0, x, W_edge, W_h, b_h, W_o1, W_o2, b_o)


@jax.jit
def kernel(x, edge_index, edge_attr, W_in, b_in, W_edge, W_h, b_h, W_o, b_o):
  src = edge_index[0].astype(jnp.int32)
  dst = edge_index[1].astype(jnp.int32)
  pad = E_PAD - E
  zpad = jnp.zeros((pad,), jnp.int32)
  src2d = jnp.concatenate([src, zpad]).reshape(IDX_ROWS, CHUNK)
  dst2d = jnp.concatenate([dst, zpad]).reshape(IDX_ROWS, CHUNK)

  b_in2 = b_in.reshape(1, HIDDEN)
  b_h2 = b_h.reshape(1, HIDDEN)
  b_o2 = b_o.reshape(1, HIDDEN)

  h0 = _tc_h0(x, W_in, b_in2)
  acc = _gather_segsum(h0, src2d, dst2d)
  ea2 = _edge_segsum(edge_attr, dst2d)
  h = _tc_step(acc, ea2, h0, W_edge, W_h, b_h2)

  acc = _gather_segsum(h, src2d, dst2d)
  h = _tc_step(acc, ea2, h0, W_edge, W_h, b_h2)

  acc = _gather_segsum(h, src2d, dst2d)
  return _tc_step_out(acc, ea2, h0, x, W_edge, W_h, b_h2,
                      W_o[:D_ATOM], W_o[D_ATOM:], b_o2)
